# baseline (speedup 1.0000x reference)
import functools

import jax
import jax.numpy as jnp
from jax import lax
from jax.experimental import pallas as pl
from jax.experimental.pallas import tpu as pltpu
from jax.experimental.pallas import tpu_sc as plsc

EMB = 64
NC = 2
NS = 16
NW = NC * NS
CHUNK = 128

V = 1000000
NB_FULL = V // 128
TAIL = V - NB_FULL * 128

_params = pltpu.CompilerParams(
    use_tc_tiling_on_sc=True, needs_layout_passes=False)


@functools.lru_cache(maxsize=None)
def _make_repack():
    mesh = plsc.VectorSubcoreMesh(core_axis_name="c", subcore_axis_name="s")
    n_iter = (NB_FULL + NW - 1) // NW
    assert n_iter % 2 == 1

    @functools.partial(
        pl.kernel,
        mesh=mesh,
        out_type=jax.ShapeDtypeStruct((V // 2, 128), jnp.float32),
        scratch_types=[
            pltpu.VMEM((EMB, 128), jnp.float32),
            pltpu.VMEM((EMB, 128), jnp.float32),
            pltpu.VMEM((EMB, 128), jnp.float32),
            pltpu.SemaphoreType.DMA,
            pltpu.SemaphoreType.DMA,
        ],
        compiler_params=_params,
    )
    def k1(tT_hbm, t_tail_hbm, t2_hbm, in_a, in_b, out_v, sem_a, sem_b):
        wid = lax.axis_index("s") * NC + lax.axis_index("c")
        iot = lax.iota(jnp.int32, 16)

        def fire(b, buf, sem):
            pltpu.async_copy(tT_hbm.at[:, pl.ds(b * 128, 128)], buf, sem)

        def wait(b, buf, sem):
            pltpu.make_async_copy(
                tT_hbm.at[:, pl.ds(b * 128, 128)], buf, sem).wait()

        def permute_write(b, buf):
            for g in range(8):
                rowids = iot + 16 * (g % 4)
                colids = jnp.full((16,), g // 4, jnp.int32)
                two = jnp.full((16,), 2, jnp.int32)
                for k in range(EMB):
                    out_v[k, pl.ds(16 * g, 16)] = plsc.load_gather(
                        buf, [rowids, colids])
                    colids = colids + two
            pltpu.sync_copy(out_v, t2_hbm.at[pl.ds(b * EMB, EMB)])

        fire(wid, in_a, sem_a)

        def body(i, carry):
            b0 = wid + NW * (2 * i)
            b1 = b0 + NW
            b2 = b1 + NW

            @pl.when(b1 < NB_FULL)
            def _():
                fire(b1, in_b, sem_b)

            @pl.when(b0 < NB_FULL)
            def _():
                wait(b0, in_a, sem_a)
                permute_write(b0, in_a)

            @pl.when(b2 < NB_FULL)
            def _():
                fire(b2, in_a, sem_a)

            @pl.when(b1 < NB_FULL)
            def _():
                wait(b1, in_b, sem_b)
                permute_write(b1, in_b)

            return carry

        lax.fori_loop(0, (n_iter + 1) // 2, body, 0)

        @pl.when(wid == 0)
        def _():
            pltpu.sync_copy(t_tail_hbm, out_v.at[pl.ds(0, TAIL // 2)])
            pltpu.sync_copy(out_v.at[pl.ds(0, TAIL // 2)],
                            t2_hbm.at[pl.ds(NB_FULL * EMB, TAIL // 2)])

    return k1


@functools.lru_cache(maxsize=None)
def _make_gather(batch: int, n_fields: int):
    mesh = plsc.VectorSubcoreMesh(core_axis_name="c", subcore_axis_name="s")
    n_blk = batch // CHUNK
    blk_per_w = n_blk // NW
    assert n_blk % NW == 0 and n_fields % 2 == 0

    @functools.partial(
        pl.kernel,
        mesh=mesh,
        out_type=jax.ShapeDtypeStruct((n_fields, EMB, batch), jnp.float32),
        scratch_types=[
            pltpu.VMEM((n_fields, CHUNK), jnp.int32),
            pltpu.VMEM((CHUNK,), jnp.int32),
            pltpu.VMEM((CHUNK,), jnp.int32),
            pltpu.VMEM((CHUNK, 128), jnp.float32),
            pltpu.VMEM((CHUNK, 128), jnp.float32),
            pltpu.VMEM((EMB, CHUNK), jnp.float32),
            pltpu.SemaphoreType.DMA,
            pltpu.SemaphoreType.DMA,
        ],
        compiler_params=_params,
    )
    def k2(yT_hbm, t2_hbm, out_hbm,
           y_v, idx_a, idx_b, rows_a, rows_b, out_v, sem_a, sem_b):
        wid = lax.axis_index("s") * NC + lax.axis_index("c")
        iot = lax.iota(jnp.int32, 16)

        def prep(f, ibuf):
            for g in range(8):
                v = y_v[f, pl.ds(16 * g, 16)]
                ibuf[pl.ds(16 * g, 16)] = lax.shift_right_logical(v, 1)

        def fire(ibuf, rbuf, sem):
            pltpu.async_copy(t2_hbm.at[ibuf], rbuf, sem)

        def wait(rbuf, sem):
            pltpu.make_async_copy(
                t2_hbm.at[pl.ds(0, CHUNK)], rbuf, sem).wait()

        def trans_write(f, b0, rbuf):
            one = jnp.full((16,), 1, jnp.int32)
            for g in range(8):
                vy = y_v[f, pl.ds(16 * g, 16)]
                col = lax.bitwise_and(vy, 1) * EMB
                rowids = iot + 16 * g
                for c2 in range(EMB):
                    out_v[c2, pl.ds(16 * g, 16)] = plsc.load_gather(
                        rbuf, [rowids, col])
                    col = col + one
            pltpu.sync_copy(out_v, out_hbm.at[f, :, pl.ds(b0, CHUNK)])

        def body_b(bi, carry):
            b0 = (wid * blk_per_w + bi) * CHUNK
            pltpu.sync_copy(yT_hbm.at[:, pl.ds(b0, CHUNK)], y_v)
            prep(0, idx_a)
            fire(idx_a, rows_a, sem_a)

            def body_f(fi, carry2):
                f0 = 2 * fi
                f1 = f0 + 1
                prep(f1, idx_b)
                fire(idx_b, rows_b, sem_b)
                wait(rows_a, sem_a)
                trans_write(f0, b0, rows_a)

                @pl.when(f0 + 2 < n_fields)
                def _():
                    prep(f0 + 2, idx_a)
                    fire(idx_a, rows_a, sem_a)

                wait(rows_b, sem_b)
                trans_write(f1, b0, rows_b)
                return carry2

            lax.fori_loop(0, n_fields // 2, body_f, 0)
            return carry

        lax.fori_loop(0, blk_per_w, body_b, 0)

    return k2


def kernel(y, table):
    batch, n_fields = y.shape
    yT = jnp.transpose(y.astype(jnp.int32))
    tT = jnp.transpose(table)
    t_tail = jnp.reshape(table[V - TAIL:, :], (TAIL // 2, 128))
    t2 = _make_repack()(tT, t_tail)
    out = _make_gather(batch, n_fields)(yT, t2)
    return jnp.transpose(out, (2, 0, 1))


# speedup vs baseline: 2.7070x; 2.7070x over previous
import functools

import jax
import jax.numpy as jnp
from jax import lax
from jax.experimental import pallas as pl
from jax.experimental.pallas import tpu as pltpu
from jax.experimental.pallas import tpu_sc as plsc

EMB = 64
NC = 2
NS = 16
NW = NC * NS
CHUNK = 128


NACC = 4
GROUP_ROWS = NACC * CHUNK


@functools.lru_cache(maxsize=None)
def _make_gather(B: int, V: int):
    b_per_w = B // NW
    n_chunks = b_per_w // CHUNK
    n_groups = n_chunks // NACC
    assert n_chunks % NACC == 0 and n_groups % 2 == 0
    mesh = plsc.VectorSubcoreMesh(core_axis_name="c", subcore_axis_name="s")

    @functools.partial(
        pl.kernel,
        mesh=mesh,
        out_type=jax.ShapeDtypeStruct((B, EMB), jnp.float32),
        scratch_types=[
            pltpu.VMEM((n_chunks, CHUNK), jnp.int32),
            pltpu.VMEM((GROUP_ROWS, EMB), jnp.float32),
            pltpu.VMEM((GROUP_ROWS, EMB), jnp.float32),
            pltpu.SemaphoreType.DMA,
            pltpu.SemaphoreType.DMA,
        ],
        compiler_params=pltpu.CompilerParams(use_tc_tiling_on_sc=False),
    )
    def k(idx_hbm, table_hbm, out_hbm, idx_v, buf0, buf1, sem0, sem1):
        wid = lax.axis_index("s") * NC + lax.axis_index("c")
        pltpu.sync_copy(idx_hbm.at[wid], idx_v)
        base = wid * b_per_w

        def fire(g, buf, sem):
            for b in range(NACC):
                pltpu.async_copy(table_hbm.at[idx_v.at[g * NACC + b]],
                                 buf.at[pl.ds(b * CHUNK, CHUNK)], sem)

        def drain_write(g, buf, sem):
            for b in range(NACC):
                pltpu.make_async_copy(table_hbm.at[idx_v.at[g * NACC + b]],
                                      buf.at[pl.ds(b * CHUNK, CHUNK)], sem).wait()
            pltpu.sync_copy(buf, out_hbm.at[pl.ds(base + g * GROUP_ROWS,
                                                  GROUP_ROWS)])

        fire(0, buf0, sem0)

        def body(g2, carry):
            g0 = 2 * g2
            fire(g0 + 1, buf1, sem1)
            drain_write(g0, buf0, sem0)

            @pl.when(g0 + 2 < n_groups)
            def _():
                fire(g0 + 2, buf0, sem0)

            drain_write(g0 + 1, buf1, sem1)
            return carry

        lax.fori_loop(0, n_groups // 2, body, 0)

    return k


def kernel(y, table):
    batch, n_fields = y.shape
    B = batch * n_fields
    idx = y.astype(jnp.int32).reshape(NW, B // NW // CHUNK, CHUNK)
    out = _make_gather(B, table.shape[0])(idx, table)
    return out.reshape(batch, n_fields, EMB)
